# Initial kernel scaffold; baseline (speedup 1.0000x reference)
import jax
import jax.numpy as jnp
from jax.experimental import pallas as pl


def kernel(x, edge_index, W1, b1, W2, b2):
    raise NotImplementedError("write your pallas kernel here")


# speedup vs baseline: 28.4421x; 28.4421x over previous
import functools

import jax
import jax.numpy as jnp
from jax import lax
from jax.experimental import pallas as pl
from jax.experimental.pallas import tpu as pltpu
from jax.experimental.pallas import tpu_sc as plsc

N_NODES = 10000
N_EDGES = 320000
D_FEAT = 128
HIDDEN = 16

NC, NS = 2, 16
NW = NC * NS
LANE = 128
ROWS_PER_W = 80
E_PAD = NW * ROWS_PER_W * LANE
NP = 10240
RPS = NP // NS
ROW_BLK = 512
GRID = NP // ROW_BLK

_mesh = plsc.VectorSubcoreMesh(core_axis_name="c", subcore_axis_name="s")


def _worker_id():
    return lax.axis_index("s") * NC + lax.axis_index("c")


@functools.partial(
    pl.kernel,
    out_type=jax.ShapeDtypeStruct((NC, NP), jnp.float32),
    mesh=_mesh,
    scratch_types=[
        pltpu.VMEM((ROWS_PER_W, LANE), jnp.int32),
        pltpu.VMEM((LANE,), jnp.float32),
        pltpu.VMEM((RPS,), jnp.float32),
        pltpu.VMEM_SHARED((NP,), jnp.float32),
        pltpu.SemaphoreType.DMA,
    ],
)
def _deg_kernel(dst_hbm, out_hbm, didx_v, ones_v, stg_v, acc_sh, sem):
    c = lax.axis_index("c")
    s = lax.axis_index("s")
    w = _worker_id()

    def fill_ones(i, _):
        ones_v[pl.ds(i * 16, 16)] = jnp.full((16,), 1.0, jnp.float32)
        return 0

    lax.fori_loop(0, LANE // 16, fill_ones, 0)

    def fill_zero(i, _):
        stg_v[pl.ds(i * 16, 16)] = jnp.zeros((16,), jnp.float32)
        return 0

    lax.fori_loop(0, RPS // 16, fill_zero, 0)
    pltpu.sync_copy(stg_v, acc_sh.at[pl.ds(s * RPS, RPS)])
    plsc.subcore_barrier()

    pltpu.async_copy(dst_hbm.at[pl.ds(w * ROWS_PER_W, ROWS_PER_W)], didx_v, sem).wait()

    def step(j, _):
        pltpu.sync_copy(ones_v, acc_sh.at[didx_v.at[j]], add=True)
        return 0

    lax.fori_loop(0, ROWS_PER_W, step, 0)
    plsc.subcore_barrier()
    pltpu.sync_copy(acc_sh.at[pl.ds(s * RPS, RPS)], stg_v)
    pltpu.sync_copy(stg_v, out_hbm.at[c, pl.ds(s * RPS, RPS)])


@functools.partial(
    pl.kernel,
    out_type=jax.ShapeDtypeStruct((NC, NP, HIDDEN), jnp.float32),
    mesh=_mesh,
    scratch_types=[
        pltpu.VMEM((ROWS_PER_W, LANE), jnp.int32),
        pltpu.VMEM((ROWS_PER_W, LANE), jnp.int32),
        pltpu.VMEM((LANE, HIDDEN), jnp.float32),
        pltpu.VMEM((RPS, HIDDEN), jnp.float32),
        pltpu.VMEM_SHARED((NP, HIDDEN), jnp.float32),
        pltpu.SemaphoreType.DMA,
        pltpu.SemaphoreType.DMA,
    ],
    compiler_params=pltpu.CompilerParams(use_tc_tiling_on_sc=False),
)
def _agg_kernel(g_hbm, src_hbm, dst_hbm, out_hbm,
                sidx_v, didx_v, rows_v, stg_v, acc_sh, sem_i, sem_g):
    c = lax.axis_index("c")
    s = lax.axis_index("s")
    w = _worker_id()

    def fill_zero(i, _):
        stg_v[i, :] = jnp.zeros((16,), jnp.float32)
        return 0

    lax.fori_loop(0, RPS, fill_zero, 0)
    pltpu.sync_copy(stg_v, acc_sh.at[pl.ds(s * RPS, RPS)])
    plsc.subcore_barrier()

    cp_s = pltpu.async_copy(src_hbm.at[pl.ds(w * ROWS_PER_W, ROWS_PER_W)], sidx_v, sem_i)
    cp_d = pltpu.async_copy(dst_hbm.at[pl.ds(w * ROWS_PER_W, ROWS_PER_W)], didx_v, sem_i)
    cp_s.wait()
    cp_d.wait()

    def step(j, _):
        pltpu.async_copy(g_hbm.at[sidx_v.at[j]], rows_v, sem_g).wait()
        pltpu.sync_copy(rows_v, acc_sh.at[didx_v.at[j]], add=True)
        return 0

    lax.fori_loop(0, ROWS_PER_W, step, 0)
    plsc.subcore_barrier()
    pltpu.sync_copy(acc_sh.at[pl.ds(s * RPS, RPS)], stg_v)
    pltpu.sync_copy(stg_v, out_hbm.at[c, pl.ds(s * RPS, RPS)])


def _tc1_body(x_ref, w1_ref, dp_ref, g1_ref):
    deg = dp_ref[0, :] + dp_ref[1, :] + 1.0
    dinv = lax.rsqrt(deg)
    h = jnp.dot(x_ref[...], w1_ref[...], preferred_element_type=jnp.float32)
    g1_ref[...] = h * dinv[:, None]


def _tc2_body(agg_ref, g1_ref, dp_ref, b1_ref, w2_ref, g2_ref):
    dinv = lax.rsqrt(dp_ref[0, :] + dp_ref[1, :] + 1.0)
    a = agg_ref[0] + agg_ref[1] + g1_ref[...]
    h = jnp.maximum(a * dinv[:, None] + b1_ref[...][None, :], 0.0)
    h2 = jnp.dot(h, w2_ref[...], preferred_element_type=jnp.float32)
    g2_ref[...] = h2 * dinv[:, None]


def _tc3_body(agg_ref, g2_ref, dp_ref, b2_ref, out_ref):
    dinv = lax.rsqrt(dp_ref[0, :] + dp_ref[1, :] + 1.0)
    a = agg_ref[0] + agg_ref[1] + g2_ref[...]
    o = a * dinv[:, None] + b2_ref[...][None, :]
    m = jnp.max(o, axis=1, keepdims=True)
    e = jnp.exp(o - m)
    out_ref[...] = (o - m) - jnp.log(jnp.sum(e, axis=1, keepdims=True))


def _row_spec():
    return pl.BlockSpec((ROW_BLK, HIDDEN), lambda i: (i, 0))


def _dp_spec():
    return pl.BlockSpec((NC, ROW_BLK), lambda i: (0, i))


def _agg_spec():
    return pl.BlockSpec((NC, ROW_BLK, HIDDEN), lambda i: (0, i, 0))


def kernel(x, edge_index, W1, b1, W2, b2):
    ei = edge_index.astype(jnp.int32)
    pad = E_PAD - N_EDGES
    src = jnp.concatenate([ei[0], jnp.full((pad,), N_NODES, jnp.int32)])
    dst = jnp.concatenate([ei[1], jnp.full((pad,), N_NODES, jnp.int32)])
    src2d = src.reshape(NW * ROWS_PER_W, LANE)
    dst2d = dst.reshape(NW * ROWS_PER_W, LANE)
    xp = jnp.pad(x, ((0, NP - N_NODES), (0, 0)))

    dp = _deg_kernel(dst2d)

    g1 = pl.pallas_call(
        _tc1_body,
        grid=(GRID,),
        in_specs=[
            pl.BlockSpec((ROW_BLK, D_FEAT), lambda i: (i, 0)),
            pl.BlockSpec((D_FEAT, HIDDEN), lambda i: (0, 0)),
            _dp_spec(),
        ],
        out_specs=_row_spec(),
        out_shape=jax.ShapeDtypeStruct((NP, HIDDEN), jnp.float32),
    )(xp, W1, dp)

    agg1 = _agg_kernel(g1, src2d, dst2d)

    g2 = pl.pallas_call(
        _tc2_body,
        grid=(GRID,),
        in_specs=[
            _agg_spec(),
            _row_spec(),
            _dp_spec(),
            pl.BlockSpec((HIDDEN,), lambda i: (0,)),
            pl.BlockSpec((HIDDEN, HIDDEN), lambda i: (0, 0)),
        ],
        out_specs=_row_spec(),
        out_shape=jax.ShapeDtypeStruct((NP, HIDDEN), jnp.float32),
    )(agg1, g1, dp, b1, W2)

    agg2 = _agg_kernel(g2, src2d, dst2d)

    out = pl.pallas_call(
        _tc3_body,
        grid=(GRID,),
        in_specs=[
            _agg_spec(),
            _row_spec(),
            _dp_spec(),
            pl.BlockSpec((HIDDEN,), lambda i: (0,)),
        ],
        out_specs=_row_spec(),
        out_shape=jax.ShapeDtypeStruct((NP, HIDDEN), jnp.float32),
    )(agg2, g2, dp, b2)

    return out[:N_NODES]


# speedup vs baseline: 45.4457x; 1.5978x over previous
import functools

import jax
import jax.numpy as jnp
from jax import lax
from jax.experimental import pallas as pl
from jax.experimental.pallas import tpu as pltpu
from jax.experimental.pallas import tpu_sc as plsc

N_NODES = 10000
N_EDGES = 320000
D_FEAT = 128
HIDDEN = 16

NC, NS = 2, 16
NW = NC * NS
LANE = 128
ROWS_PER_W = 80
E_PAD = NW * ROWS_PER_W * LANE
NP = 10240
RPS = NP // NS
ROW_BLK = 512
GRID = NP // ROW_BLK

_mesh = plsc.VectorSubcoreMesh(core_axis_name="c", subcore_axis_name="s")


def _worker_id():
    return lax.axis_index("s") * NC + lax.axis_index("c")


@functools.partial(
    pl.kernel,
    out_type=jax.ShapeDtypeStruct((NC, NP), jnp.float32),
    mesh=_mesh,
    scratch_types=[
        pltpu.VMEM((ROWS_PER_W, LANE), jnp.int32),
        pltpu.VMEM((LANE,), jnp.float32),
        pltpu.VMEM((RPS,), jnp.float32),
        pltpu.VMEM_SHARED((NP,), jnp.float32),
        pltpu.SemaphoreType.DMA,
    ],
)
def _deg_kernel(dst_hbm, out_hbm, didx_v, ones_v, stg_v, acc_sh, sem):
    c = lax.axis_index("c")
    s = lax.axis_index("s")
    w = _worker_id()

    def fill_ones(i, _):
        ones_v[pl.ds(i * 16, 16)] = jnp.full((16,), 1.0, jnp.float32)
        return 0

    lax.fori_loop(0, LANE // 16, fill_ones, 0)

    def fill_zero(i, _):
        stg_v[pl.ds(i * 16, 16)] = jnp.zeros((16,), jnp.float32)
        return 0

    lax.fori_loop(0, RPS // 16, fill_zero, 0)
    pltpu.sync_copy(stg_v, acc_sh.at[pl.ds(s * RPS, RPS)])
    plsc.subcore_barrier()

    pltpu.async_copy(dst_hbm.at[pl.ds(w * ROWS_PER_W, ROWS_PER_W)], didx_v, sem).wait()

    def step(j, _):
        pltpu.sync_copy(ones_v, acc_sh.at[didx_v.at[j]], add=True)
        return 0

    lax.fori_loop(0, ROWS_PER_W, step, 0)
    plsc.subcore_barrier()
    pltpu.sync_copy(acc_sh.at[pl.ds(s * RPS, RPS)], stg_v)
    pltpu.sync_copy(stg_v, out_hbm.at[c, pl.ds(s * RPS, RPS)])


@functools.partial(
    pl.kernel,
    out_type=jax.ShapeDtypeStruct((NC, NP, HIDDEN), jnp.float32),
    mesh=_mesh,
    scratch_types=[
        pltpu.VMEM((ROWS_PER_W, LANE), jnp.int32),
        pltpu.VMEM((ROWS_PER_W, LANE), jnp.int32),
        pltpu.VMEM((LANE, HIDDEN), jnp.float32),
        pltpu.VMEM((LANE, HIDDEN), jnp.float32),
        pltpu.VMEM((RPS, HIDDEN), jnp.float32),
        pltpu.VMEM_SHARED((NP, HIDDEN), jnp.float32),
        pltpu.SemaphoreType.DMA,
        pltpu.SemaphoreType.DMA,
        pltpu.SemaphoreType.DMA,
    ],
    compiler_params=pltpu.CompilerParams(use_tc_tiling_on_sc=False),
)
def _agg_kernel(g_hbm, src_hbm, dst_hbm, out_hbm,
                sidx_v, didx_v, rows0_v, rows1_v, stg_v, acc_sh,
                sem_i, sem_g0, sem_g1):
    c = lax.axis_index("c")
    s = lax.axis_index("s")
    w = _worker_id()

    def fill_zero(i, _):
        stg_v[i, :] = jnp.zeros((16,), jnp.float32)
        return 0

    lax.fori_loop(0, RPS, fill_zero, 0)
    pltpu.sync_copy(stg_v, acc_sh.at[pl.ds(s * RPS, RPS)])
    plsc.subcore_barrier()

    cp_s = pltpu.async_copy(src_hbm.at[pl.ds(w * ROWS_PER_W, ROWS_PER_W)], sidx_v, sem_i)
    cp_d = pltpu.async_copy(dst_hbm.at[pl.ds(w * ROWS_PER_W, ROWS_PER_W)], didx_v, sem_i)
    cp_s.wait()
    cp_d.wait()

    pltpu.async_copy(g_hbm.at[sidx_v.at[0]], rows0_v, sem_g0)

    def step(i, _):
        j0 = 2 * i
        pltpu.async_copy(g_hbm.at[sidx_v.at[j0 + 1]], rows1_v, sem_g1)
        pltpu.make_async_copy(g_hbm.at[sidx_v.at[j0]], rows0_v, sem_g0).wait()
        pltpu.sync_copy(rows0_v, acc_sh.at[didx_v.at[j0]], add=True)

        @pl.when(j0 + 2 < ROWS_PER_W)
        def _():
            pltpu.async_copy(g_hbm.at[sidx_v.at[j0 + 2]], rows0_v, sem_g0)

        pltpu.make_async_copy(g_hbm.at[sidx_v.at[j0 + 1]], rows1_v, sem_g1).wait()
        pltpu.sync_copy(rows1_v, acc_sh.at[didx_v.at[j0 + 1]], add=True)
        return 0

    lax.fori_loop(0, ROWS_PER_W // 2, step, 0)
    plsc.subcore_barrier()
    pltpu.sync_copy(acc_sh.at[pl.ds(s * RPS, RPS)], stg_v)
    pltpu.sync_copy(stg_v, out_hbm.at[c, pl.ds(s * RPS, RPS)])


def _tc1_body(x_ref, w1_ref, dp_ref, g1_ref):
    deg = dp_ref[0, :] + dp_ref[1, :] + 1.0
    dinv = lax.rsqrt(deg)
    h = jnp.dot(x_ref[...], w1_ref[...], preferred_element_type=jnp.float32)
    g1_ref[...] = h * dinv[:, None]


def _tc2_body(agg_ref, g1_ref, dp_ref, b1_ref, w2_ref, g2_ref):
    dinv = lax.rsqrt(dp_ref[0, :] + dp_ref[1, :] + 1.0)
    a = agg_ref[0] + agg_ref[1] + g1_ref[...]
    h = jnp.maximum(a * dinv[:, None] + b1_ref[...][None, :], 0.0)
    h2 = jnp.dot(h, w2_ref[...], preferred_element_type=jnp.float32)
    g2_ref[...] = h2 * dinv[:, None]


def _tc3_body(agg_ref, g2_ref, dp_ref, b2_ref, out_ref):
    dinv = lax.rsqrt(dp_ref[0, :] + dp_ref[1, :] + 1.0)
    a = agg_ref[0] + agg_ref[1] + g2_ref[...]
    o = a * dinv[:, None] + b2_ref[...][None, :]
    m = jnp.max(o, axis=1, keepdims=True)
    e = jnp.exp(o - m)
    out_ref[...] = (o - m) - jnp.log(jnp.sum(e, axis=1, keepdims=True))


def _row_spec():
    return pl.BlockSpec((ROW_BLK, HIDDEN), lambda i: (i, 0))


def _dp_spec():
    return pl.BlockSpec((NC, ROW_BLK), lambda i: (0, i))


def _agg_spec():
    return pl.BlockSpec((NC, ROW_BLK, HIDDEN), lambda i: (0, i, 0))


def kernel(x, edge_index, W1, b1, W2, b2):
    ei = edge_index.astype(jnp.int32)
    pad = E_PAD - N_EDGES
    pad_idx = N_NODES + jnp.arange(pad, dtype=jnp.int32) % (NP - N_NODES)
    src = jnp.concatenate([ei[0], pad_idx])
    dst = jnp.concatenate([ei[1], pad_idx])
    src2d = src.reshape(NW * ROWS_PER_W, LANE)
    dst2d = dst.reshape(NW * ROWS_PER_W, LANE)
    xp = jnp.pad(x, ((0, NP - N_NODES), (0, 0)))

    dp = _deg_kernel(dst2d)

    g1 = pl.pallas_call(
        _tc1_body,
        grid=(GRID,),
        in_specs=[
            pl.BlockSpec((ROW_BLK, D_FEAT), lambda i: (i, 0)),
            pl.BlockSpec((D_FEAT, HIDDEN), lambda i: (0, 0)),
            _dp_spec(),
        ],
        out_specs=_row_spec(),
        out_shape=jax.ShapeDtypeStruct((NP, HIDDEN), jnp.float32),
    )(xp, W1, dp)

    agg1 = _agg_kernel(g1, src2d, dst2d)

    g2 = pl.pallas_call(
        _tc2_body,
        grid=(GRID,),
        in_specs=[
            _agg_spec(),
            _row_spec(),
            _dp_spec(),
            pl.BlockSpec((HIDDEN,), lambda i: (0,)),
            pl.BlockSpec((HIDDEN, HIDDEN), lambda i: (0, 0)),
        ],
        out_specs=_row_spec(),
        out_shape=jax.ShapeDtypeStruct((NP, HIDDEN), jnp.float32),
    )(agg1, g1, dp, b1, W2)

    agg2 = _agg_kernel(g2, src2d, dst2d)

    out = pl.pallas_call(
        _tc3_body,
        grid=(GRID,),
        in_specs=[
            _agg_spec(),
            _row_spec(),
            _dp_spec(),
            pl.BlockSpec((HIDDEN,), lambda i: (0,)),
        ],
        out_specs=_row_spec(),
        out_shape=jax.ShapeDtypeStruct((NP, HIDDEN), jnp.float32),
    )(agg2, g2, dp, b2)

    return out[:N_NODES]


# speedup vs baseline: 53.5177x; 1.1776x over previous
import functools

import jax
import jax.numpy as jnp
from jax import lax
from jax.experimental import pallas as pl
from jax.experimental.pallas import tpu as pltpu
from jax.experimental.pallas import tpu_sc as plsc

N_NODES = 10000
N_EDGES = 320000
D_FEAT = 128
HIDDEN = 16

NC, NS = 2, 16
NW = NC * NS
LANE = 128
ER = N_EDGES // LANE
BASE_ROWS = ER // NW
EXTRA = ER - BASE_ROWS * NW
MAXR = BASE_ROWS + 1
NP = 10240
RPS = NP // NS

_mesh = plsc.VectorSubcoreMesh(core_axis_name="c", subcore_axis_name="s")
_sc_params = pltpu.CompilerParams(use_tc_tiling_on_sc=False)


def _worker_split():
    c = lax.axis_index("c")
    s = lax.axis_index("s")
    w = s * NC + c
    nrows = BASE_ROWS + jnp.where(w < EXTRA, 1, 0)
    base = BASE_ROWS * w + jnp.minimum(w, EXTRA)
    return c, s, w, nrows, base


@functools.partial(
    pl.kernel,
    out_type=jax.ShapeDtypeStruct((NC, NP), jnp.float32),
    mesh=_mesh,
    scratch_types=[
        pltpu.VMEM((MAXR, LANE), jnp.int32),
        pltpu.VMEM((LANE,), jnp.float32),
        pltpu.VMEM((RPS,), jnp.float32),
        pltpu.VMEM_SHARED((NP,), jnp.float32),
        pltpu.SemaphoreType.DMA,
    ],
    compiler_params=_sc_params,
)
def _deg_kernel(ei_hbm, out_hbm, didx_v, ones_v, stg_v, acc_sh, sem):
    c, s, w, nrows, base = _worker_split()

    def fill_ones(i, _):
        ones_v[pl.ds(i * 16, 16)] = jnp.full((16,), 1.0, jnp.float32)
        return 0

    lax.fori_loop(0, LANE // 16, fill_ones, 0)

    def fill_zero(i, _):
        stg_v[pl.ds(i * 16, 16)] = jnp.zeros((16,), jnp.float32)
        return 0

    lax.fori_loop(0, RPS // 16, fill_zero, 0)
    pltpu.sync_copy(stg_v, acc_sh.at[pl.ds(s * RPS, RPS)])
    plsc.subcore_barrier()

    pltpu.async_copy(ei_hbm.at[1, pl.ds(base, BASE_ROWS)],
                     didx_v.at[pl.ds(0, BASE_ROWS)], sem).wait()

    @pl.when(w < EXTRA)
    def _():
        pltpu.sync_copy(ei_hbm.at[1, base + BASE_ROWS], didx_v.at[BASE_ROWS])

    def step(j, _):
        pltpu.sync_copy(ones_v, acc_sh.at[didx_v.at[j]], add=True)
        return 0

    lax.fori_loop(0, nrows, step, 0)
    plsc.subcore_barrier()
    pltpu.sync_copy(acc_sh.at[pl.ds(s * RPS, RPS)], stg_v)
    pltpu.sync_copy(stg_v, out_hbm.at[c, pl.ds(s * RPS, RPS)])


@functools.partial(
    pl.kernel,
    out_type=jax.ShapeDtypeStruct((NC, NP, HIDDEN), jnp.float32),
    mesh=_mesh,
    scratch_types=[
        pltpu.VMEM((MAXR, LANE), jnp.int32),
        pltpu.VMEM((MAXR, LANE), jnp.int32),
        pltpu.VMEM((LANE, HIDDEN), jnp.float32),
        pltpu.VMEM((LANE, HIDDEN), jnp.float32),
        pltpu.VMEM((RPS, HIDDEN), jnp.float32),
        pltpu.VMEM_SHARED((NP, HIDDEN), jnp.float32),
        pltpu.SemaphoreType.DMA,
        pltpu.SemaphoreType.DMA,
        pltpu.SemaphoreType.DMA,
    ],
    compiler_params=_sc_params,
)
def _agg_kernel(g_hbm, ei_hbm, out_hbm,
                sidx_v, didx_v, rows0_v, rows1_v, stg_v, acc_sh,
                sem_i, sem_g0, sem_g1):
    c, s, w, nrows, base = _worker_split()

    def fill_zero(i, _):
        stg_v[i, :] = jnp.zeros((16,), jnp.float32)
        return 0

    lax.fori_loop(0, RPS, fill_zero, 0)
    pltpu.sync_copy(stg_v, acc_sh.at[pl.ds(s * RPS, RPS)])
    plsc.subcore_barrier()

    cp_s = pltpu.async_copy(ei_hbm.at[0, pl.ds(base, BASE_ROWS)],
                            sidx_v.at[pl.ds(0, BASE_ROWS)], sem_i)
    cp_d = pltpu.async_copy(ei_hbm.at[1, pl.ds(base, BASE_ROWS)],
                            didx_v.at[pl.ds(0, BASE_ROWS)], sem_i)
    cp_s.wait()
    cp_d.wait()

    @pl.when(w < EXTRA)
    def _():
        pltpu.sync_copy(ei_hbm.at[0, base + BASE_ROWS], sidx_v.at[BASE_ROWS])
        pltpu.sync_copy(ei_hbm.at[1, base + BASE_ROWS], didx_v.at[BASE_ROWS])

    pltpu.async_copy(g_hbm.at[sidx_v.at[0]], rows0_v, sem_g0)

    def step(i, _):
        j0 = 2 * i
        pltpu.async_copy(g_hbm.at[sidx_v.at[j0 + 1]], rows1_v, sem_g1)
        pltpu.make_async_copy(g_hbm.at[sidx_v.at[j0]], rows0_v, sem_g0).wait()
        pltpu.sync_copy(rows0_v, acc_sh.at[didx_v.at[j0]], add=True)

        @pl.when(j0 + 2 < nrows)
        def _():
            pltpu.async_copy(g_hbm.at[sidx_v.at[j0 + 2]], rows0_v, sem_g0)

        pltpu.make_async_copy(g_hbm.at[sidx_v.at[j0 + 1]], rows1_v, sem_g1).wait()
        pltpu.sync_copy(rows1_v, acc_sh.at[didx_v.at[j0 + 1]], add=True)
        return 0

    lax.fori_loop(0, BASE_ROWS // 2, step, 0)

    @pl.when(nrows > BASE_ROWS)
    def _():
        pltpu.make_async_copy(g_hbm.at[sidx_v.at[BASE_ROWS]], rows0_v, sem_g0).wait()
        pltpu.sync_copy(rows0_v, acc_sh.at[didx_v.at[BASE_ROWS]], add=True)

    plsc.subcore_barrier()
    pltpu.sync_copy(acc_sh.at[pl.ds(s * RPS, RPS)], stg_v)
    pltpu.sync_copy(stg_v, out_hbm.at[c, pl.ds(s * RPS, RPS)])


def _dinv(dp_ref):
    return lax.rsqrt(dp_ref[0, :N_NODES] + dp_ref[1, :N_NODES] + 1.0)


def _tc1_body(x_ref, w1_ref, dp_ref, g1_ref):
    h = jnp.dot(x_ref[...], w1_ref[...], preferred_element_type=jnp.float32)
    g1_ref[...] = h * _dinv(dp_ref)[:, None]


def _tc2_body(agg_ref, g1_ref, dp_ref, b1_ref, w2_ref, g2_ref):
    dinv = _dinv(dp_ref)
    a = agg_ref[0, :N_NODES] + agg_ref[1, :N_NODES] + g1_ref[...]
    h = jnp.maximum(a * dinv[:, None] + b1_ref[...], 0.0)
    h2 = jnp.dot(h, w2_ref[...], preferred_element_type=jnp.float32)
    g2_ref[...] = h2 * dinv[:, None]


def _tc3_body(agg_ref, g2_ref, dp_ref, b2_ref, out_ref):
    dinv = _dinv(dp_ref)
    a = agg_ref[0, :N_NODES] + agg_ref[1, :N_NODES] + g2_ref[...]
    o = a * dinv[:, None] + b2_ref[...]
    m = jnp.max(o, axis=1, keepdims=True)
    e = jnp.exp(o - m)
    out_ref[...] = (o - m) - jnp.log(jnp.sum(e, axis=1, keepdims=True))


_ROWS_F32 = jax.ShapeDtypeStruct((N_NODES, HIDDEN), jnp.float32)


def kernel(x, edge_index, W1, b1, W2, b2):
    ei = edge_index.astype(jnp.int32).reshape(2, ER, LANE)
    b1r = b1.reshape(1, HIDDEN)
    b2r = b2.reshape(1, HIDDEN)

    dp = _deg_kernel(ei)
    g1 = pl.pallas_call(_tc1_body, out_shape=_ROWS_F32)(x, W1, dp)
    agg1 = _agg_kernel(g1, ei)
    g2 = pl.pallas_call(_tc2_body, out_shape=_ROWS_F32)(agg1, g1, dp, b1r, W2)
    agg2 = _agg_kernel(g2, ei)
    out = pl.pallas_call(_tc3_body, out_shape=_ROWS_F32)(agg2, g2, dp, b2r)
    return out


# speedup vs baseline: 60.1362x; 1.1237x over previous
import functools

import jax
import jax.numpy as jnp
from jax import lax
from jax.experimental import pallas as pl
from jax.experimental.pallas import tpu as pltpu
from jax.experimental.pallas import tpu_sc as plsc

N_NODES = 10000
N_EDGES = 320000
D_FEAT = 128
HIDDEN = 16

NC, NS = 2, 16
NW = NC * NS
LANE = 128
ER = N_EDGES // LANE
BASE_ROWS = ER // NW
EXTRA = ER - BASE_ROWS * NW
MAXR = BASE_ROWS + 1
CH = 6
NCH = BASE_ROWS // CH
NP = 10240
RPS = NP // NS

_mesh = plsc.VectorSubcoreMesh(core_axis_name="c", subcore_axis_name="s")
_sc_params = pltpu.CompilerParams(use_tc_tiling_on_sc=False)


def _worker_split():
    c = lax.axis_index("c")
    s = lax.axis_index("s")
    w = s * NC + c
    nrows = BASE_ROWS + jnp.where(w < EXTRA, 1, 0)
    base = BASE_ROWS * w + jnp.minimum(w, EXTRA)
    return c, s, w, nrows, base


@functools.partial(
    pl.kernel,
    out_type=jax.ShapeDtypeStruct((NC, NP), jnp.float32),
    mesh=_mesh,
    scratch_types=[
        pltpu.VMEM((MAXR, LANE), jnp.int32),
        pltpu.VMEM((LANE,), jnp.float32),
        pltpu.VMEM((RPS,), jnp.float32),
        pltpu.VMEM_SHARED((NP,), jnp.float32),
        pltpu.SemaphoreType.DMA,
    ],
    compiler_params=_sc_params,
)
def _deg_kernel(ei_hbm, out_hbm, didx_v, ones_v, stg_v, acc_sh, sem):
    c, s, w, nrows, base = _worker_split()

    def fill_ones(i, _):
        ones_v[pl.ds(i * 16, 16)] = jnp.full((16,), 1.0, jnp.float32)
        return 0

    lax.fori_loop(0, LANE // 16, fill_ones, 0)

    def fill_zero(i, _):
        stg_v[pl.ds(i * 16, 16)] = jnp.zeros((16,), jnp.float32)
        return 0

    lax.fori_loop(0, RPS // 16, fill_zero, 0)
    pltpu.sync_copy(stg_v, acc_sh.at[pl.ds(s * RPS, RPS)])
    plsc.subcore_barrier()

    pltpu.async_copy(ei_hbm.at[1, pl.ds(base, BASE_ROWS)],
                     didx_v.at[pl.ds(0, BASE_ROWS)], sem).wait()

    @pl.when(w < EXTRA)
    def _():
        pltpu.sync_copy(ei_hbm.at[1, base + BASE_ROWS], didx_v.at[BASE_ROWS])

    def step(j, _):
        pltpu.sync_copy(ones_v, acc_sh.at[didx_v.at[j]], add=True)
        return 0

    lax.fori_loop(0, nrows, step, 0)
    plsc.subcore_barrier()
    pltpu.sync_copy(acc_sh.at[pl.ds(s * RPS, RPS)], stg_v)
    pltpu.sync_copy(stg_v, out_hbm.at[c, pl.ds(s * RPS, RPS)])


@functools.partial(
    pl.kernel,
    out_type=jax.ShapeDtypeStruct((NC, NP, D_FEAT), jnp.float32),
    mesh=_mesh,
    scratch_types=[
        pltpu.VMEM((MAXR, LANE), jnp.int32),
        pltpu.VMEM((MAXR, LANE), jnp.int32),
        pltpu.VMEM((LANE, HIDDEN), jnp.float32),
        pltpu.VMEM((LANE, HIDDEN), jnp.float32),
        pltpu.VMEM((RPS, HIDDEN), jnp.float32),
        pltpu.VMEM_SHARED((NP, HIDDEN), jnp.float32),
        pltpu.SemaphoreType.DMA,
        pltpu.SemaphoreType.DMA,
        pltpu.SemaphoreType.DMA,
    ],
    compiler_params=_sc_params,
)
def _agg_kernel(g_hbm, ei_hbm, out_hbm,
                sidx_v, didx_v, rows0_v, rows1_v, stg_v, acc_sh,
                sem_i, sem_g0, sem_g1):
    c, s, w, nrows, base = _worker_split()

    def fill_zero(i, _):
        stg_v[i, :] = jnp.zeros((16,), jnp.float32)
        return 0

    lax.fori_loop(0, RPS, fill_zero, 0)
    pltpu.sync_copy(stg_v, acc_sh.at[pl.ds(s * RPS, RPS)])
    plsc.subcore_barrier()

    cp_s = pltpu.async_copy(ei_hbm.at[0, pl.ds(base, BASE_ROWS)],
                            sidx_v.at[pl.ds(0, BASE_ROWS)], sem_i)
    cp_d = pltpu.async_copy(ei_hbm.at[1, pl.ds(base, BASE_ROWS)],
                            didx_v.at[pl.ds(0, BASE_ROWS)], sem_i)
    cp_s.wait()
    cp_d.wait()

    @pl.when(w < EXTRA)
    def _():
        pltpu.sync_copy(ei_hbm.at[0, base + BASE_ROWS], sidx_v.at[BASE_ROWS])
        pltpu.sync_copy(ei_hbm.at[1, base + BASE_ROWS], didx_v.at[BASE_ROWS])

    def scale(j, _):
        r = j // 8
        k = j % 8
        v = sidx_v[r, pl.ds(k * 16, 16)]
        sidx_v[r, pl.ds(k * 16, 16)] = v * 8
        return 0

    lax.fori_loop(0, MAXR * 8, scale, 0)

    pltpu.async_copy(g_hbm.at[sidx_v.at[0]], rows0_v, sem_g0)

    def step(i, _):
        j0 = 2 * i
        pltpu.async_copy(g_hbm.at[sidx_v.at[j0 + 1]], rows1_v, sem_g1)
        pltpu.make_async_copy(g_hbm.at[sidx_v.at[j0]], rows0_v, sem_g0).wait()
        pltpu.sync_copy(rows0_v, acc_sh.at[didx_v.at[j0]], add=True)

        @pl.when(j0 + 2 < nrows)
        def _():
            pltpu.async_copy(g_hbm.at[sidx_v.at[j0 + 2]], rows0_v, sem_g0)

        pltpu.make_async_copy(g_hbm.at[sidx_v.at[j0 + 1]], rows1_v, sem_g1).wait()
        pltpu.sync_copy(rows1_v, acc_sh.at[didx_v.at[j0 + 1]], add=True)
        return 0

    lax.fori_loop(0, BASE_ROWS // 2, step, 0)

    @pl.when(nrows > BASE_ROWS)
    def _():
        pltpu.make_async_copy(g_hbm.at[sidx_v.at[BASE_ROWS]], rows0_v, sem_g0).wait()
        pltpu.sync_copy(rows0_v, acc_sh.at[didx_v.at[BASE_ROWS]], add=True)

    plsc.subcore_barrier()
    pltpu.sync_copy(acc_sh.at[pl.ds(s * RPS, RPS)], stg_v)
    pltpu.sync_copy(stg_v, out_hbm.at[c, pl.ds(s * RPS, RPS), pl.ds(0, HIDDEN)])


def _dinv(dp_ref):
    return lax.rsqrt(dp_ref[0, :N_NODES] + dp_ref[1, :N_NODES] + 1.0)


def _tc1_body(x_ref, w1_ref, dp_ref, g1_ref):
    h = jnp.dot(x_ref[...], w1_ref[...], preferred_element_type=jnp.float32)
    g1_ref[:, pl.ds(0, HIDDEN)] = h * _dinv(dp_ref)[:, None]


def _tc2_body(agg_ref, g1_ref, dp_ref, b1_ref, w2_ref, g2_ref):
    dinv = _dinv(dp_ref)
    a = (agg_ref[0, :N_NODES, :HIDDEN] + agg_ref[1, :N_NODES, :HIDDEN]
         + g1_ref[:, :HIDDEN])
    h = jnp.maximum(a * dinv[:, None] + b1_ref[...], 0.0)
    h2 = jnp.dot(h, w2_ref[...], preferred_element_type=jnp.float32)
    g2_ref[:, pl.ds(0, HIDDEN)] = h2 * dinv[:, None]


def _tc3_body(agg_ref, g2_ref, dp_ref, b2_ref, out_ref):
    dinv = _dinv(dp_ref)
    a = (agg_ref[0, :N_NODES, :HIDDEN] + agg_ref[1, :N_NODES, :HIDDEN]
         + g2_ref[:, :HIDDEN])
    o = a * dinv[:, None] + b2_ref[...]
    m = jnp.max(o, axis=1, keepdims=True)
    e = jnp.exp(o - m)
    out_ref[...] = (o - m) - jnp.log(jnp.sum(e, axis=1, keepdims=True))


_WIDE_F32 = jax.ShapeDtypeStruct((N_NODES, D_FEAT), jnp.float32)


def kernel(x, edge_index, W1, b1, W2, b2):
    ei = edge_index.astype(jnp.int32).reshape(2, ER, LANE)
    b1r = b1.reshape(1, HIDDEN)
    b2r = b2.reshape(1, HIDDEN)

    dp = _deg_kernel(ei)
    g1 = pl.pallas_call(_tc1_body, out_shape=_WIDE_F32)(x, W1, dp)
    agg1 = _agg_kernel(g1.reshape(N_NODES * 8, HIDDEN), ei)
    g2 = pl.pallas_call(_tc2_body, out_shape=_WIDE_F32)(agg1, g1, dp, b1r, W2)
    agg2 = _agg_kernel(g2.reshape(N_NODES * 8, HIDDEN), ei)
    out = pl.pallas_call(
        _tc3_body,
        out_shape=jax.ShapeDtypeStruct((N_NODES, HIDDEN), jnp.float32),
    )(agg2, g2, dp, b2r)
    return out
